# Initial kernel scaffold; baseline (speedup 1.0000x reference)
import jax
import jax.numpy as jnp
from jax.experimental import pallas as pl


def kernel(queries, keys, datastore_vals, k):
    raise NotImplementedError("write your pallas kernel here")


# speedup vs baseline: 1.5187x; 1.5187x over previous
import functools

import jax
import jax.numpy as jnp
from jax import lax
from jax.experimental import pallas as pl

Q = 64
D = 64
N_KEYS = 1000000
K = 16
VOCAB = 100000
KNN_TEMP = 10.0

BLK = 25000
NB = N_KEYS // BLK
VBLK = 12800
NVB = (VOCAB + VBLK - 1) // VBLK

NEG = -3.4e38


def _dist_topk_kernel(q_ref, kb_ref, vals_ref, idx_ref):
    b = pl.program_id(0)
    q = q_ref[...]
    kb = kb_ref[...]
    k2 = jnp.sum(kb * kb, axis=1)
    s = 2.0 * lax.dot_general(
        q, kb, (((1,), (1,)), ((), ())),
        preferred_element_type=jnp.float32) - k2[None, :]
    iota = lax.broadcasted_iota(jnp.int32, s.shape, 1)
    x = s
    tvs, tis = [], []
    for _ in range(K):
        m = jnp.max(x, axis=1, keepdims=True)
        pos = jnp.min(jnp.where(x >= m, iota, BLK), axis=1, keepdims=True)
        tvs.append(m[:, 0])
        tis.append(pos[:, 0])
        x = jnp.where(iota == pos, NEG, x)
    vals_ref[0] = jnp.stack(tvs, axis=1)
    idx_ref[0] = jnp.stack(tis, axis=1) + b * BLK


def _merge_kernel(cv_ref, ci_ref, w_ref, gi_ref):
    v = cv_ref[...]
    gidx = ci_ref[...]
    c = v.shape[1]
    iota = lax.broadcasted_iota(jnp.int32, v.shape, 1)
    x = v
    tvs, tis = [], []
    for _ in range(K):
        m = jnp.max(x, axis=1, keepdims=True)
        pos = jnp.min(jnp.where(x >= m, iota, c), axis=1, keepdims=True)
        sel = iota == pos
        tis.append(jnp.sum(jnp.where(sel, gidx, 0), axis=1))
        tvs.append(m[:, 0])
        x = jnp.where(sel, NEG, x)
    tv = jnp.stack(tvs, axis=1)
    ti = jnp.stack(tis, axis=1)
    e = jnp.exp((tv - tv[:, 0:1]) / KNN_TEMP)
    w_ref[...] = e / jnp.sum(e, axis=1, keepdims=True)
    gi_ref[...] = ti


def _scatter_kernel(w_ref, tok_ref, out_ref):
    pid = pl.program_id(0)
    w = w_ref[...]
    tok = tok_ref[...]
    cols = pid * VBLK + lax.broadcasted_iota(jnp.int32, (Q, VBLK), 1)
    acc = jnp.zeros((Q, VBLK), jnp.float32)
    for j in range(K):
        acc += jnp.where(tok[:, j:j + 1] == cols, w[:, j:j + 1], 0.0)
    out_ref[...] = acc


@functools.partial(jax.jit, static_argnames=())
def _run(queries, keys, datastore_vals):
    vals, idx = pl.pallas_call(
        _dist_topk_kernel,
        grid=(NB,),
        in_specs=[
            pl.BlockSpec((Q, D), lambda b: (0, 0)),
            pl.BlockSpec((BLK, D), lambda b: (b, 0)),
        ],
        out_specs=[
            pl.BlockSpec((1, Q, K), lambda b: (b, 0, 0)),
            pl.BlockSpec((1, Q, K), lambda b: (b, 0, 0)),
        ],
        out_shape=[
            jax.ShapeDtypeStruct((NB, Q, K), jnp.float32),
            jax.ShapeDtypeStruct((NB, Q, K), jnp.int32),
        ],
    )(queries, keys)

    cv = vals.transpose(1, 0, 2).reshape(Q, NB * K)
    ci = idx.transpose(1, 0, 2).reshape(Q, NB * K)

    w, gi = pl.pallas_call(
        _merge_kernel,
        out_shape=[
            jax.ShapeDtypeStruct((Q, K), jnp.float32),
            jax.ShapeDtypeStruct((Q, K), jnp.int32),
        ],
    )(cv, ci)

    tok = jnp.take(datastore_vals, gi, axis=0)

    probs = pl.pallas_call(
        _scatter_kernel,
        grid=(NVB,),
        in_specs=[
            pl.BlockSpec((Q, K), lambda b: (0, 0)),
            pl.BlockSpec((Q, K), lambda b: (0, 0)),
        ],
        out_specs=pl.BlockSpec((Q, VBLK), lambda b: (0, b)),
        out_shape=jax.ShapeDtypeStruct((Q, VOCAB), jnp.float32),
    )(w, tok)
    return probs


def kernel(queries, keys, datastore_vals, k):
    del k
    return _run(queries, keys, datastore_vals)


# speedup vs baseline: 1.8005x; 1.1855x over previous
import functools

import jax
import jax.numpy as jnp
from jax import lax
from jax.experimental import pallas as pl

Q = 64
D = 64
N_KEYS = 1000000
K = 16
VOCAB = 100000
KNN_TEMP = 10.0

BLK1 = 20000
NB1 = N_KEYS // BLK1
GRP = 32
C1 = BLK1 // GRP
NG = NB1 * C1

BLK2 = 5000
NB2 = N_KEYS // BLK2

VBLK = 12800
NVB = (VOCAB + VBLK - 1) // VBLK

NEG = -3.4e38
NEG_TEST = -1.0e38


def _dot_s(q, kb, k2row):
    return 2.0 * lax.dot_general(
        q, kb, (((1,), (1,)), ((), ())),
        preferred_element_type=jnp.float32) - k2row


def _gmax_kernel(q_ref, kb_ref, gm_ref, k2_ref):
    q = q_ref[...]
    kb = kb_ref[...]
    k2 = jnp.sum(kb * kb, axis=1)
    k2row = k2.reshape(1, BLK1)
    s = _dot_s(q, kb, k2row)
    f = s
    w = BLK1
    while w > C1:
        w //= 2
        f = jnp.maximum(f[:, :w], f[:, w:])
    gm_ref[0] = f
    k2_ref[0] = k2row


def _tau_kernel(gm_ref, tau_ref):
    x = gm_ref[...]
    m = jnp.max(x, axis=1, keepdims=True)
    for _ in range(K - 1):
        x = jnp.where(x >= m, NEG, x)
        m = jnp.max(x, axis=1, keepdims=True)
    tau_ref[...] = m


def _rescan_kernel(q_ref, kb_ref, k2_ref, tau_ref, cv_ref, ci_ref):
    q = q_ref[...]
    kb = kb_ref[...]
    tau = tau_ref[...]
    s = _dot_s(q, kb, k2_ref[0])
    iota = lax.broadcasted_iota(jnp.int32, s.shape, 1)
    iota16 = lax.broadcasted_iota(jnp.int32, (Q, K), 1)
    m0 = jnp.max(s, axis=1, keepdims=True)

    def cond(carry):
        t, m, x, cv, ci = carry
        return jnp.logical_and(t < K, jnp.max(m - tau) >= 0.0)

    def body(carry):
        t, m, x, cv, ci = carry
        pos = jnp.min(jnp.where(x >= m, iota, BLK2), axis=1, keepdims=True)
        cv = jnp.where(iota16 == t, m, cv)
        ci = jnp.where(iota16 == t, pos, ci)
        x = jnp.where(iota == pos, NEG, x)
        m = jnp.max(x, axis=1, keepdims=True)
        return t + 1, m, x, cv, ci

    init = (jnp.int32(0), m0, s,
            jnp.full((Q, K), NEG, jnp.float32), jnp.zeros((Q, K), jnp.int32))
    _, _, _, cv, ci = lax.while_loop(cond, body, init)
    cv_ref[0] = cv
    ci_ref[0] = ci + pl.program_id(0) * BLK2


def _merge_kernel(cv_ref, ci_ref, w_ref, gi_ref):
    v = cv_ref[...]
    gidx = ci_ref[...]
    c = v.shape[1]
    iota = lax.broadcasted_iota(jnp.int32, v.shape, 1)
    x = v
    tvs, tis = [], []
    for _ in range(K):
        m = jnp.max(x, axis=1, keepdims=True)
        pos = jnp.min(jnp.where(x >= m, iota, c), axis=1, keepdims=True)
        sel = iota == pos
        tis.append(jnp.sum(jnp.where(sel, gidx, 0), axis=1))
        tvs.append(m[:, 0])
        x = jnp.where(sel, NEG, x)
    tv = jnp.stack(tvs, axis=1)
    ti = jnp.stack(tis, axis=1)
    e = jnp.exp((tv - tv[:, 0:1]) / KNN_TEMP)
    w_ref[...] = e / jnp.sum(e, axis=1, keepdims=True)
    gi_ref[...] = ti


def _scatter_kernel(w_ref, tok_ref, out_ref):
    pid = pl.program_id(0)
    w = w_ref[...]
    tok = tok_ref[...]
    cols = pid * VBLK + lax.broadcasted_iota(jnp.int32, (Q, VBLK), 1)
    acc = jnp.zeros((Q, VBLK), jnp.float32)
    for j in range(K):
        acc += jnp.where(tok[:, j:j + 1] == cols, w[:, j:j + 1], 0.0)
    out_ref[...] = acc


@jax.jit
def _run(queries, keys, datastore_vals):
    gm, k2 = pl.pallas_call(
        _gmax_kernel,
        grid=(NB1,),
        in_specs=[
            pl.BlockSpec((Q, D), lambda b: (0, 0)),
            pl.BlockSpec((BLK1, D), lambda b: (b, 0)),
        ],
        out_specs=[
            pl.BlockSpec((1, Q, C1), lambda b: (b, 0, 0)),
            pl.BlockSpec((1, 1, BLK1), lambda b: (b, 0, 0)),
        ],
        out_shape=[
            jax.ShapeDtypeStruct((NB1, Q, C1), jnp.float32),
            jax.ShapeDtypeStruct((NB1, 1, BLK1), jnp.float32),
        ],
    )(queries, keys)

    gm2 = gm.transpose(1, 0, 2).reshape(Q, NG)
    k2r = k2.reshape(NB2, 1, BLK2)

    tau = pl.pallas_call(
        _tau_kernel,
        out_shape=jax.ShapeDtypeStruct((Q, 1), jnp.float32),
    )(gm2)

    cv, ci = pl.pallas_call(
        _rescan_kernel,
        grid=(NB2,),
        in_specs=[
            pl.BlockSpec((Q, D), lambda b: (0, 0)),
            pl.BlockSpec((BLK2, D), lambda b: (b, 0)),
            pl.BlockSpec((1, 1, BLK2), lambda b: (b, 0, 0)),
            pl.BlockSpec((Q, 1), lambda b: (0, 0)),
        ],
        out_specs=[
            pl.BlockSpec((1, Q, K), lambda b: (b, 0, 0)),
            pl.BlockSpec((1, Q, K), lambda b: (b, 0, 0)),
        ],
        out_shape=[
            jax.ShapeDtypeStruct((NB2, Q, K), jnp.float32),
            jax.ShapeDtypeStruct((NB2, Q, K), jnp.int32),
        ],
    )(queries, keys, k2r, tau)

    cvf = cv.transpose(1, 0, 2).reshape(Q, NB2 * K)
    cif = ci.transpose(1, 0, 2).reshape(Q, NB2 * K)

    w, gi = pl.pallas_call(
        _merge_kernel,
        out_shape=[
            jax.ShapeDtypeStruct((Q, K), jnp.float32),
            jax.ShapeDtypeStruct((Q, K), jnp.int32),
        ],
    )(cvf, cif)

    tok = jnp.take(datastore_vals, gi, axis=0)

    probs = pl.pallas_call(
        _scatter_kernel,
        grid=(NVB,),
        in_specs=[
            pl.BlockSpec((Q, K), lambda b: (0, 0)),
            pl.BlockSpec((Q, K), lambda b: (0, 0)),
        ],
        out_specs=pl.BlockSpec((Q, VBLK), lambda b: (0, b)),
        out_shape=jax.ShapeDtypeStruct((Q, VOCAB), jnp.float32),
    )(w, tok)
    return probs


def kernel(queries, keys, datastore_vals, k):
    del k
    return _run(queries, keys, datastore_vals)


# speedup vs baseline: 2.4601x; 1.3664x over previous
import jax
import jax.numpy as jnp
from jax import lax
from jax.experimental import pallas as pl
from jax.experimental.pallas import tpu as pltpu

Q = 64
D = 64
N_KEYS = 1000000
K = 16
VOCAB = 100000
KNN_TEMP = 10.0

BLK1 = 20000
NB1 = N_KEYS // BLK1
GRP = 32
C1 = BLK1 // GRP

BLK2 = 10000
NB2 = N_KEYS // BLK2

VBLK = 12800
NVB = (VOCAB + VBLK - 1) // VBLK

NEG = -3.4e38


def _pass1_kernel(q_ref, kb_ref, tau_ref, k2_ref, r_ref):
    b = pl.program_id(0)

    @pl.when(b == 0)
    def _():
        r_ref[...] = jnp.full((Q, K), NEG, jnp.float32)

    q2 = q_ref[...] * 2.0
    kb = kb_ref[...]
    sq = kb * kb
    ones = jnp.ones((1, D), jnp.float32)
    k2row = lax.dot_general(
        ones, sq, (((1,), (1,)), ((), ())),
        preferred_element_type=jnp.float32)
    s = lax.dot_general(
        q2, kb, (((1,), (1,)), ((), ())),
        preferred_element_type=jnp.float32) - k2row
    f = s
    w = BLK1
    while w > C1:
        w //= 2
        f = jnp.maximum(f[:, :w], f[:, w:])
    iota16 = lax.broadcasted_iota(jnp.int32, (Q, K), 1)
    x = jnp.concatenate([r_ref[...], f], axis=1)
    rn = r_ref[...]
    for t in range(K):
        m = jnp.max(x, axis=1, keepdims=True)
        rn = jnp.where(iota16 == t, m, rn)
        x = jnp.where(x >= m, NEG, x)
    r_ref[...] = rn
    tau_ref[...] = rn[:, K - 1:K]
    k2_ref[0:1] = k2row[:, :BLK2].reshape(1, 1, BLK2)
    k2_ref[1:2] = k2row[:, BLK2:].reshape(1, 1, BLK2)


def _rescan_kernel(q_ref, kb_ref, k2_ref, tau_ref, cv_ref, ci_ref):
    q2 = q_ref[...] * 2.0
    kb = kb_ref[...]
    tau = tau_ref[...] - 0.05
    s = lax.dot_general(
        q2, kb, (((1,), (1,)), ((), ())),
        preferred_element_type=jnp.float32) - k2_ref[0]
    iota = lax.broadcasted_iota(jnp.int32, s.shape, 1)
    iota16 = lax.broadcasted_iota(jnp.int32, (Q, K), 1)
    m0 = jnp.max(s, axis=1, keepdims=True)

    def cond(carry):
        t, m, x, cv, ci = carry
        return jnp.logical_and(t < K, jnp.max(m - tau) >= 0.0)

    def body(carry):
        t, m, x, cv, ci = carry
        pos = jnp.min(jnp.where(x >= m, iota, BLK2), axis=1, keepdims=True)
        cv = jnp.where(iota16 == t, m, cv)
        ci = jnp.where(iota16 == t, pos, ci)
        x = jnp.where(iota == pos, NEG, x)
        m = jnp.max(x, axis=1, keepdims=True)
        return t + 1, m, x, cv, ci

    init = (jnp.int32(0), m0, s,
            jnp.full((Q, K), NEG, jnp.float32), jnp.zeros((Q, K), jnp.int32))
    _, _, _, cv, ci = lax.while_loop(cond, body, init)
    ci = ci + pl.program_id(0) * BLK2
    cv_ref[...] = cv.reshape(Q, 1, 1, K)
    ci_ref[...] = ci.reshape(Q, 1, 1, K)


def _merge_kernel(cv_ref, ci_ref, w_ref, gi_ref):
    v = cv_ref[...]
    gidx = ci_ref[...]
    c = v.shape[1]
    iota = lax.broadcasted_iota(jnp.int32, v.shape, 1)
    iota16 = lax.broadcasted_iota(jnp.int32, (Q, K), 1)
    x = v
    tv = jnp.full((Q, K), NEG, jnp.float32)
    ti = jnp.zeros((Q, K), jnp.int32)
    for t in range(K):
        m = jnp.max(x, axis=1, keepdims=True)
        pos = jnp.min(jnp.where(x >= m, iota, c), axis=1, keepdims=True)
        sel = iota == pos
        gi_t = jnp.sum(jnp.where(sel, gidx, 0), axis=1, keepdims=True)
        tv = jnp.where(iota16 == t, m, tv)
        ti = jnp.where(iota16 == t, gi_t, ti)
        x = jnp.where(sel, NEG, x)
    e = jnp.exp((tv - tv[:, 0:1]) / KNN_TEMP)
    w_ref[...] = e / jnp.sum(e, axis=1, keepdims=True)
    gi_ref[...] = ti


def _scatter_kernel(w_ref, tok_ref, out_ref):
    pid = pl.program_id(0)
    w = w_ref[...]
    tok = tok_ref[...]
    cols = pid * VBLK + lax.broadcasted_iota(jnp.int32, (Q, VBLK), 1)
    acc = jnp.zeros((Q, VBLK), jnp.float32)
    for j in range(K):
        acc += jnp.where(tok[:, j:j + 1] == cols, w[:, j:j + 1], 0.0)
    out_ref[...] = acc


@jax.jit
def _run(queries, keys, datastore_vals):
    tau, k2 = pl.pallas_call(
        _pass1_kernel,
        grid=(NB1,),
        in_specs=[
            pl.BlockSpec((Q, D), lambda b: (0, 0)),
            pl.BlockSpec((BLK1, D), lambda b: (b, 0)),
        ],
        out_specs=[
            pl.BlockSpec((Q, 1), lambda b: (0, 0)),
            pl.BlockSpec((2, 1, BLK2), lambda b: (b, 0, 0)),
        ],
        out_shape=[
            jax.ShapeDtypeStruct((Q, 1), jnp.float32),
            jax.ShapeDtypeStruct((NB2, 1, BLK2), jnp.float32),
        ],
        scratch_shapes=[pltpu.VMEM((Q, K), jnp.float32)],
    )(queries, keys)

    cv, ci = pl.pallas_call(
        _rescan_kernel,
        grid=(NB2,),
        in_specs=[
            pl.BlockSpec((Q, D), lambda b: (0, 0)),
            pl.BlockSpec((BLK2, D), lambda b: (b, 0)),
            pl.BlockSpec((1, 1, BLK2), lambda b: (b, 0, 0)),
            pl.BlockSpec((Q, 1), lambda b: (0, 0)),
        ],
        out_specs=[
            pl.BlockSpec((Q, 1, 1, K), lambda b: (0, b, 0, 0)),
            pl.BlockSpec((Q, 1, 1, K), lambda b: (0, b, 0, 0)),
        ],
        out_shape=[
            jax.ShapeDtypeStruct((Q, NB2, 1, K), jnp.float32),
            jax.ShapeDtypeStruct((Q, NB2, 1, K), jnp.int32),
        ],
    )(queries, keys, k2, tau)

    cvf = cv.reshape(Q, NB2 * K)
    cif = ci.reshape(Q, NB2 * K)

    w, gi = pl.pallas_call(
        _merge_kernel,
        out_shape=[
            jax.ShapeDtypeStruct((Q, K), jnp.float32),
            jax.ShapeDtypeStruct((Q, K), jnp.int32),
        ],
    )(cvf, cif)

    tok = jnp.take(datastore_vals, gi, axis=0)

    probs = pl.pallas_call(
        _scatter_kernel,
        grid=(NVB,),
        in_specs=[
            pl.BlockSpec((Q, K), lambda b: (0, 0)),
            pl.BlockSpec((Q, K), lambda b: (0, 0)),
        ],
        out_specs=pl.BlockSpec((Q, VBLK), lambda b: (0, b)),
        out_shape=jax.ShapeDtypeStruct((Q, VOCAB), jnp.float32),
    )(w, tok)
    return probs


def kernel(queries, keys, datastore_vals, k):
    del k
    return _run(queries, keys, datastore_vals)
